# Initial kernel scaffold; baseline (speedup 1.0000x reference)
import jax
import jax.numpy as jnp
from jax.experimental import pallas as pl


def kernel(x, edge_index, W1, b1, W2, b2, W3, b3, Wp, bp):
    raise NotImplementedError("write your pallas kernel here")


# speedup vs baseline: 8.4812x; 8.4812x over previous
import functools

import jax
import jax.numpy as jnp
from jax import lax
from jax.experimental import pallas as pl
from jax.experimental.pallas import tpu as pltpu
from jax.experimental.pallas import tpu_sc as plsc

_NC = 2
_NS = 16
_NW = _NC * _NS
_CH = 80
_BN = 256


def _cdiv(a, b):
    return (a + b - 1) // b


def _sc_scatter_partials(y, src, dst, zerosH):
    N, H = y.shape
    (E,) = src.shape
    EPT = E // _NW
    n_chunks = EPT // _CH
    RPT = N // _NS
    mesh = plsc.VectorSubcoreMesh(core_axis_name="c", subcore_axis_name="s")

    @functools.partial(
        pl.kernel,
        out_type=jax.ShapeDtypeStruct((_NC, N, H), jnp.float32),
        mesh=mesh,
        scratch_types=[
            pltpu.VMEM((_CH,), jnp.int32),
            pltpu.VMEM((_CH,), jnp.int32),
            pltpu.VMEM((_CH, H), jnp.float32),
            pltpu.VMEM_SHARED((N, H), jnp.float32),
            pltpu.SemaphoreType.DMA,
        ],
    )
    def k(y_hbm, src_hbm, dst_hbm, zero_hbm, out_hbm, src_v, dst_v, rows_v, acc_sh, sem):
        c = lax.axis_index("c")
        s = lax.axis_index("s")
        wid = c * _NS + s
        r0 = s * RPT

        @pl.when(c == 0)
        def _():
            pltpu.sync_copy(y_hbm.at[pl.ds(r0, RPT)], acc_sh.at[pl.ds(r0, RPT)])

        @pl.when(c != 0)
        def _():
            pltpu.sync_copy(zero_hbm.at[pl.ds(r0, RPT)], acc_sh.at[pl.ds(r0, RPT)])

        plsc.subcore_barrier()
        base0 = wid * EPT

        def body(i, carry):
            base = base0 + i * _CH
            pltpu.sync_copy(src_hbm.at[pl.ds(base, _CH)], src_v)
            pltpu.sync_copy(dst_hbm.at[pl.ds(base, _CH)], dst_v)
            pltpu.async_copy(y_hbm.at[src_v], rows_v, sem).wait()
            pltpu.sync_copy(rows_v, acc_sh.at[dst_v], add=True)
            return carry

        lax.fori_loop(0, n_chunks, body, 0)
        plsc.subcore_barrier()
        pltpu.sync_copy(acc_sh.at[pl.ds(r0, RPT)], out_hbm.at[c, pl.ds(r0, RPT)])

    return k(y, src, dst, zerosH)


def _tc_dinv(d0, d1):
    N, H = d0.shape

    def body(d0_ref, d1_ref, o_ref):
        deg = d0_ref[:, :1] + d1_ref[:, :1]
        o_ref[...] = lax.rsqrt(deg)

    return pl.pallas_call(
        body,
        grid=(_cdiv(N, _BN),),
        in_specs=[
            pl.BlockSpec((_BN, H), lambda i: (i, 0)),
            pl.BlockSpec((_BN, H), lambda i: (i, 0)),
        ],
        out_specs=pl.BlockSpec((_BN, 1), lambda i: (i, 0)),
        out_shape=jax.ShapeDtypeStruct((N, 1), jnp.float32),
    )(d0, d1)


def _tc_matmul_scale(x, W, dinv):
    N, D = x.shape
    H = W.shape[1]

    def body(x_ref, w_ref, dinv_ref, o_ref):
        y = jnp.dot(x_ref[...], w_ref[...], preferred_element_type=jnp.float32)
        o_ref[...] = dinv_ref[...] * y

    return pl.pallas_call(
        body,
        grid=(_cdiv(N, _BN),),
        in_specs=[
            pl.BlockSpec((_BN, D), lambda i: (i, 0)),
            pl.BlockSpec((D, H), lambda i: (0, 0)),
            pl.BlockSpec((_BN, 1), lambda i: (i, 0)),
        ],
        out_specs=pl.BlockSpec((_BN, H), lambda i: (i, 0)),
        out_shape=jax.ShapeDtypeStruct((N, H), jnp.float32),
    )(x, W, dinv)


def _tc_combine_matmul(p0, p1, dinv, b, W, bout, scale_out):
    N, D = p0.shape
    H = W.shape[1]

    def body(p0_ref, p1_ref, dinv_ref, b_ref, w_ref, bout_ref, o_ref):
        h = dinv_ref[...] * (p0_ref[...] + p1_ref[...]) + b_ref[...]
        h = jnp.maximum(h, 0.0)
        y = jnp.dot(h, w_ref[...], preferred_element_type=jnp.float32)
        if scale_out:
            y = dinv_ref[...] * y
        o_ref[...] = y + bout_ref[...]

    return pl.pallas_call(
        body,
        grid=(_cdiv(N, _BN),),
        in_specs=[
            pl.BlockSpec((_BN, D), lambda i: (i, 0)),
            pl.BlockSpec((_BN, D), lambda i: (i, 0)),
            pl.BlockSpec((_BN, 1), lambda i: (i, 0)),
            pl.BlockSpec((1, D), lambda i: (0, 0)),
            pl.BlockSpec((D, H), lambda i: (0, 0)),
            pl.BlockSpec((1, H), lambda i: (0, 0)),
        ],
        out_specs=pl.BlockSpec((_BN, H), lambda i: (i, 0)),
        out_shape=jax.ShapeDtypeStruct((N, H), jnp.float32),
    )(p0, p1, dinv, b, W, bout)


def kernel(x, edge_index, W1, b1, W2, b2, W3, b3, Wp, bp):
    N, D = x.shape
    NP = _cdiv(N, _NS * 8) * _NS * 8
    xp = jnp.pad(x, ((0, NP - N), (0, 0)))
    src = edge_index[0]
    dst = edge_index[1]

    zerosH = jnp.zeros((NP, D), jnp.float32)
    onesH = jnp.ones((NP, D), jnp.float32)

    dpart = _sc_scatter_partials(onesH, dst, dst, zerosH)
    dinv = _tc_dinv(dpart[0], dpart[1])

    zH = jnp.zeros((1, W2.shape[1]), jnp.float32)
    y = _tc_matmul_scale(xp, W1, dinv)
    p = _sc_scatter_partials(y, src, dst, zerosH)
    y = _tc_combine_matmul(p[0], p[1], dinv, b1.reshape(1, -1), W2, zH, True)
    p = _sc_scatter_partials(y, src, dst, zerosH)
    y = _tc_combine_matmul(p[0], p[1], dinv, b2.reshape(1, -1), W3, zH, True)
    p = _sc_scatter_partials(y, src, dst, zerosH)
    out = _tc_combine_matmul(p[0], p[1], dinv, b3.reshape(1, -1), Wp,
                             bp.reshape(1, -1), False)
    return out[:N]
